# baseline (speedup 1.0000x reference)
import functools

import jax
import jax.numpy as jnp
from jax import lax
from jax.experimental import pallas as pl
from jax.experimental.pallas import tpu as pltpu
from jax.experimental.pallas import tpu_sc as plsc

RES = 100
D = 256
LANES = 16
MAGIC = 12582912.0
K = 128
SPC = 16
FK = K * SPC


def _encoder_kernel(n_rows, n_workers):
    per_w = n_rows // n_workers
    n_chunks = per_w // FK
    mesh = plsc.VectorSubcoreMesh(core_axis_name="c", subcore_axis_name="s")

    @functools.partial(
        pl.kernel,
        mesh=mesh,
        out_type=jax.ShapeDtypeStruct((n_rows, D), jnp.float32),
        scratch_types=[
            pltpu.VMEM((FK,), jnp.float32),
            pltpu.VMEM((SPC, K), jnp.int32),
            pltpu.VMEM((2, K, D), jnp.float32),
            pltpu.SemaphoreType.DMA,
        ],
    )
    def body(frac_hbm, pe_hbm, out_hbm, frac_v, idx_v, rows_v, sem):
        wid = lax.axis_index("s") * 2 + lax.axis_index("c")
        base = wid * per_w

        def chunk(c, carry):
            off = base + c * FK
            pltpu.sync_copy(frac_hbm.at[pl.ds(off, FK)], frac_v)
            for j in range(FK // LANES):
                s, w = divmod(j, K // LANES)
                v = frac_v[pl.ds(j * LANES, LANES)]
                t = jnp.maximum(v, jnp.float32(1.0 / RES)) * jnp.float32(RES)
                r = (t + jnp.float32(MAGIC)) - jnp.float32(MAGIC)
                idx_v[s, pl.ds(w * LANES, LANES)] = r.astype(jnp.int32) - 1
            cps = [None, None]
            cps[0] = pltpu.async_copy(pe_hbm.at[idx_v.at[0]], rows_v.at[0], sem)
            for s in range(SPC):
                b = s % 2
                cps[b].wait()
                if s + 1 < SPC:
                    cps[1 - b] = pltpu.async_copy(
                        pe_hbm.at[idx_v.at[s + 1]], rows_v.at[1 - b], sem
                    )
                pltpu.sync_copy(rows_v.at[b], out_hbm.at[pl.ds(off + s * K, K)])
            return carry

        lax.fori_loop(0, n_chunks, chunk, 0)

    return body


def kernel(frac, pe):
    b, s = frac.shape
    n_rows = b * s
    out = _encoder_kernel(n_rows, 32)(frac.reshape(n_rows), pe)
    return out.reshape(b, s, D)


# speedup vs baseline: 2.7852x; 2.7852x over previous
import functools

import jax
import jax.numpy as jnp
from jax import lax
from jax.experimental import pallas as pl
from jax.experimental.pallas import tpu as pltpu
from jax.experimental.pallas import tpu_sc as plsc

RES = 100
D = 256
LANES = 16
MAGIC = 12582912.0
K = 128
SPC = 16
FK = K * SPC
NREP = 32


def _encoder_kernel(n_rows, n_workers):
    per_w = n_rows // n_workers
    n_chunks = per_w // FK
    mesh = plsc.VectorSubcoreMesh(core_axis_name="c", subcore_axis_name="s")

    @functools.partial(
        pl.kernel,
        mesh=mesh,
        out_type=jax.ShapeDtypeStruct((n_rows, D), jnp.float32),
        scratch_types=[
            pltpu.VMEM((FK,), jnp.float32),
            pltpu.VMEM((SPC, K), jnp.int32),
            pltpu.VMEM((2, K, D), jnp.float32),
            pltpu.SemaphoreType.DMA,
        ],
    )
    def body(frac_hbm, pe_hbm, out_hbm, frac_v, idx_v, rows_v, sem):
        wid = lax.axis_index("s") * 2 + lax.axis_index("c")
        base = wid * per_w
        rep_off = (wid % NREP) * RES

        def chunk(c, carry):
            off = base + c * FK
            pltpu.sync_copy(frac_hbm.at[pl.ds(off, FK)], frac_v)
            for j in range(FK // LANES):
                s, w = divmod(j, K // LANES)
                v = frac_v[pl.ds(j * LANES, LANES)]
                t = jnp.maximum(v, jnp.float32(1.0 / RES)) * jnp.float32(RES)
                r = (t + jnp.float32(MAGIC)) - jnp.float32(MAGIC)
                idx_v[s, pl.ds(w * LANES, LANES)] = (
                    r.astype(jnp.int32) - 1 + rep_off
                )
            cps = [None, None]
            cps[0] = pltpu.async_copy(pe_hbm.at[idx_v.at[0]], rows_v.at[0], sem)
            for s in range(SPC):
                b = s % 2
                cps[b].wait()
                if s + 1 < SPC:
                    cps[1 - b] = pltpu.async_copy(
                        pe_hbm.at[idx_v.at[s + 1]], rows_v.at[1 - b], sem
                    )
                pltpu.sync_copy(rows_v.at[b], out_hbm.at[pl.ds(off + s * K, K)])
            return carry

        lax.fori_loop(0, n_chunks, chunk, 0)

    return body


def kernel(frac, pe):
    b, s = frac.shape
    n_rows = b * s
    pe_rep = jnp.tile(pe, (NREP, 1))
    out = _encoder_kernel(n_rows, 32)(frac.reshape(n_rows), pe_rep)
    return out.reshape(b, s, D)


# speedup vs baseline: 2.8760x; 1.0326x over previous
import functools

import jax
import jax.numpy as jnp
from jax import lax
from jax.experimental import pallas as pl
from jax.experimental.pallas import tpu as pltpu
from jax.experimental.pallas import tpu_sc as plsc

RES = 100
D = 256
LANES = 16
MAGIC = 12582912.0
K = 128
SPC = 16
FK = K * SPC
NREP = 32
NBUF = 3


def _encoder_kernel(n_rows, n_workers):
    per_w = n_rows // n_workers
    n_chunks = per_w // FK
    mesh = plsc.VectorSubcoreMesh(core_axis_name="c", subcore_axis_name="s")

    @functools.partial(
        pl.kernel,
        mesh=mesh,
        out_type=jax.ShapeDtypeStruct((n_rows, D), jnp.float32),
        scratch_types=[
            pltpu.VMEM((FK,), jnp.float32),
            pltpu.VMEM((SPC, K), jnp.int32),
            pltpu.VMEM((NBUF, K, D), jnp.float32),
            pltpu.SemaphoreType.DMA,
        ],
    )
    def body(frac_hbm, pe_hbm, out_hbm, frac_v, idx_v, rows_v, sem):
        wid = lax.axis_index("s") * 2 + lax.axis_index("c")
        base = wid * per_w
        rep_off = (wid % NREP) * RES

        def chunk(c, carry):
            off = base + c * FK
            pltpu.sync_copy(frac_hbm.at[pl.ds(off, FK)], frac_v)
            for j in range(FK // LANES):
                s, w = divmod(j, K // LANES)
                v = frac_v[pl.ds(j * LANES, LANES)]
                t = jnp.maximum(v, jnp.float32(1.0 / RES)) * jnp.float32(RES)
                r = (t + jnp.float32(MAGIC)) - jnp.float32(MAGIC)
                idx_v[s, pl.ds(w * LANES, LANES)] = (
                    r.astype(jnp.int32) - 1 + rep_off
                )
            cps = [None] * SPC
            for s in range(NBUF - 1):
                cps[s] = pltpu.async_copy(
                    pe_hbm.at[idx_v.at[s]], rows_v.at[s], sem
                )
            for s in range(SPC):
                b = s % NBUF
                cps[s].wait()
                nxt = s + NBUF - 1
                if nxt < SPC:
                    cps[nxt] = pltpu.async_copy(
                        pe_hbm.at[idx_v.at[nxt]], rows_v.at[nxt % NBUF], sem
                    )
                pltpu.sync_copy(rows_v.at[b], out_hbm.at[pl.ds(off + s * K, K)])
            return carry

        lax.fori_loop(0, n_chunks, chunk, 0)

    return body


def kernel(frac, pe):
    b, s = frac.shape
    n_rows = b * s
    pe_rep = jnp.tile(pe, (NREP, 1))
    out = _encoder_kernel(n_rows, 32)(frac.reshape(n_rows), pe_rep)
    return out.reshape(b, s, D)
